# baseline (speedup 1.0000x reference)
import functools

import jax
import jax.numpy as jnp
from jax import lax
from jax.experimental import pallas as pl
from jax.experimental.pallas import tpu as pltpu
from jax.experimental.pallas import tpu_sc as plsc

N_TOKENS = 8192
N_EXPERTS = 8
N_CORES = 2
N_SUBCORES = 16
LANES = 16


def _sc_histogram(idx_flat):
    n_bins = N_TOKENS * N_EXPERTS
    n_tiles = N_CORES * N_SUBCORES
    slots_per_tile = idx_flat.shape[0] // n_tiles
    chunk = 128
    bins_per_tile = n_bins // N_SUBCORES

    mesh = plsc.VectorSubcoreMesh(core_axis_name="c", subcore_axis_name="s")

    @functools.partial(
        pl.kernel,
        mesh=mesh,
        compiler_params=pltpu.CompilerParams(needs_layout_passes=False),
        out_type=jax.ShapeDtypeStruct((N_CORES, N_EXPERTS, N_TOKENS),
                                      jnp.float32),
        scratch_types=[
            pltpu.VMEM((slots_per_tile,), jnp.int32),
            pltpu.VMEM((slots_per_tile // chunk, chunk), jnp.int32),
            pltpu.VMEM((slots_per_tile // chunk, chunk), jnp.float32),
            pltpu.VMEM((bins_per_tile,), jnp.float32),
            pltpu.VMEM_SHARED((n_bins,), jnp.float32),
        ],
    )
    def hist_kernel(idx_hbm, out_hbm, idx_v, fidx_v, ones_v, zero_v, acc_sh):
        cid = lax.axis_index("c")
        sid = lax.axis_index("s")
        wid = cid * N_SUBCORES + sid

        pltpu.sync_copy(idx_hbm.at[pl.ds(wid * slots_per_tile, slots_per_tile)],
                        idx_v)

        zeros16 = jnp.zeros((LANES,), jnp.float32)
        ones16 = jnp.ones((LANES,), jnp.float32)
        for i in range(bins_per_tile // LANES):
            zero_v[pl.ds(i * LANES, LANES)] = zeros16
        e_base = jnp.bitwise_and(lax.iota(jnp.int32, LANES),
                                 N_EXPERTS - 1) * N_TOKENS
        for j in range(slots_per_tile // chunk):
            for k in range(chunk // LANES):
                tok = idx_v[pl.ds(j * chunk + k * LANES, LANES)]
                fidx_v[j, pl.ds(k * LANES, LANES)] = tok + e_base
                ones_v[j, pl.ds(k * LANES, LANES)] = ones16

        pltpu.sync_copy(zero_v, acc_sh.at[pl.ds(sid * bins_per_tile, bins_per_tile)])
        plsc.subcore_barrier()

        for j in range(slots_per_tile // chunk):
            pltpu.sync_copy(ones_v.at[j], acc_sh.at[fidx_v.at[j]], add=True)
        plsc.subcore_barrier()

        @pl.when(sid < N_EXPERTS)
        def _():
            pltpu.sync_copy(acc_sh.at[pl.ds(sid * N_TOKENS, N_TOKENS)],
                            out_hbm.at[cid, sid])

    return hist_kernel(idx_flat)


def _ffn_body(ratio_scale, x_ref, w1_ref, b1_ref, w2_ref, b2_ref, cnt_ref,
              iw_ref, out_ref, ratio_ref):
    x = x_ref[...]
    h = jnp.dot(x, w1_ref[...], preferred_element_type=jnp.float32) + b1_ref[...]
    h = jnp.maximum(h, 0.0)
    nnz = jnp.sum((h > 0.0).astype(jnp.float32), axis=1)
    out = jnp.dot(h, w2_ref[...], preferred_element_type=jnp.float32) + b2_ref[...]
    cnt = (cnt_ref[0] + cnt_ref[1]).T
    tw = jnp.sum(cnt * iw_ref[...], axis=1)
    ctot = jnp.sum(cnt, axis=1)
    out_ref[...] = out * tw[:, None]

    @pl.when(pl.program_id(0) == 0)
    def _():
        ratio_ref[0, 0] = 0.0

    ratio_ref[0, 0] += jnp.sum(nnz * ctot) * ratio_scale


def kernel(inputs, inputs_weight, top_idx, W1, b1, W2, b2):
    n_tok, d_model = inputs.shape
    d_ff = W1.shape[1]
    cap, n_exp = top_idx.shape

    counts = _sc_histogram(top_idx.astype(jnp.int32).reshape(-1))

    blk_m = 512
    grid = (n_tok // blk_m,)
    ratio_scale = 1.0 / (n_exp * cap * d_ff)

    out, ratio = pl.pallas_call(
        functools.partial(_ffn_body, ratio_scale),
        grid=grid,
        in_specs=[
            pl.BlockSpec((blk_m, d_model), lambda i: (i, 0)),
            pl.BlockSpec((d_model, d_ff), lambda i: (0, 0)),
            pl.BlockSpec((1, d_ff), lambda i: (0, 0)),
            pl.BlockSpec((d_ff, d_model), lambda i: (0, 0)),
            pl.BlockSpec((1, d_model), lambda i: (0, 0)),
            pl.BlockSpec((2, n_exp, blk_m), lambda i: (0, 0, i)),
            pl.BlockSpec((blk_m, n_exp), lambda i: (i, 0)),
        ],
        out_specs=[
            pl.BlockSpec((blk_m, d_model), lambda i: (i, 0)),
            pl.BlockSpec((1, 1), lambda i: (0, 0), memory_space=pltpu.SMEM),
        ],
        out_shape=[
            jax.ShapeDtypeStruct((n_tok, d_model), jnp.float32),
            jax.ShapeDtypeStruct((1, 1), jnp.float32),
        ],
    )(inputs, W1, b1.reshape(1, d_ff), W2, b2.reshape(1, d_model), counts,
      inputs_weight)
    return out, ratio[0, 0]


# speedup vs baseline: 1.0035x; 1.0035x over previous
import functools

import jax
import jax.numpy as jnp
from jax import lax
from jax.experimental import pallas as pl
from jax.experimental.pallas import tpu as pltpu
from jax.experimental.pallas import tpu_sc as plsc

N_TOKENS = 8192
N_EXPERTS = 8
N_CORES = 2
N_SUBCORES = 16
LANES = 16


def _sc_histogram(idxT):
    n_bins = N_TOKENS * N_EXPERTS
    cap = idxT.shape[1]
    n_tiles = N_CORES * N_SUBCORES
    tiles_per_expert = n_tiles // N_EXPERTS
    slots_per_tile = cap // tiles_per_expert
    chunk = 128
    bins_per_tile = n_bins // N_SUBCORES

    mesh = plsc.VectorSubcoreMesh(core_axis_name="c", subcore_axis_name="s")

    @functools.partial(
        pl.kernel,
        mesh=mesh,
        compiler_params=pltpu.CompilerParams(needs_layout_passes=False),
        out_type=jax.ShapeDtypeStruct((N_CORES, N_EXPERTS, N_TOKENS),
                                      jnp.float32),
        scratch_types=[
            pltpu.VMEM((slots_per_tile,), jnp.int32),
            pltpu.VMEM((slots_per_tile // chunk, chunk), jnp.int32),
            pltpu.VMEM((slots_per_tile // chunk, chunk), jnp.float32),
            pltpu.VMEM((bins_per_tile,), jnp.float32),
            pltpu.VMEM_SHARED((n_bins,), jnp.float32),
        ],
    )
    def hist_kernel(idx_hbm, out_hbm, idx_v, fidx_v, ones_v, zero_v, acc_sh):
        cid = lax.axis_index("c")
        sid = lax.axis_index("s")
        wid = cid * N_SUBCORES + sid
        e_id = wid // tiles_per_expert
        part = wid % tiles_per_expert

        pltpu.sync_copy(
            idx_hbm.at[e_id, pl.ds(part * slots_per_tile, slots_per_tile)],
            idx_v)

        zeros16 = jnp.zeros((LANES,), jnp.float32)
        ones16 = jnp.ones((LANES,), jnp.float32)
        for i in range(bins_per_tile // LANES):
            zero_v[pl.ds(i * LANES, LANES)] = zeros16
        bin_base = e_id * N_TOKENS
        for j in range(slots_per_tile // chunk):
            for k in range(chunk // LANES):
                tok = idx_v[pl.ds(j * chunk + k * LANES, LANES)]
                fidx_v[j, pl.ds(k * LANES, LANES)] = tok + bin_base
                ones_v[j, pl.ds(k * LANES, LANES)] = ones16

        pltpu.sync_copy(zero_v, acc_sh.at[pl.ds(sid * bins_per_tile, bins_per_tile)])
        plsc.subcore_barrier()

        for j in range(slots_per_tile // chunk):
            pltpu.sync_copy(ones_v.at[j], acc_sh.at[fidx_v.at[j]], add=True)
        plsc.subcore_barrier()

        @pl.when(sid < N_EXPERTS)
        def _():
            pltpu.sync_copy(acc_sh.at[pl.ds(sid * N_TOKENS, N_TOKENS)],
                            out_hbm.at[cid, sid])

    return hist_kernel(idxT)


def _ffn_body(ratio_scale, x_ref, w1_ref, b1_ref, w2_ref, b2_ref, cnt_ref,
              iw_ref, out_ref, ratio_ref):
    x = x_ref[...]
    h = jnp.dot(x, w1_ref[...], preferred_element_type=jnp.float32) + b1_ref[...]
    h = jnp.maximum(h, 0.0)
    nnz = jnp.sum((h > 0.0).astype(jnp.float32), axis=1)
    out = jnp.dot(h, w2_ref[...], preferred_element_type=jnp.float32) + b2_ref[...]
    cnt = (cnt_ref[0] + cnt_ref[1]).T
    tw = jnp.sum(cnt * iw_ref[...], axis=1)
    ctot = jnp.sum(cnt, axis=1)
    out_ref[...] = out * tw[:, None]

    @pl.when(pl.program_id(0) == 0)
    def _():
        ratio_ref[0, 0] = 0.0

    ratio_ref[0, 0] += jnp.sum(nnz * ctot) * ratio_scale


def kernel(inputs, inputs_weight, top_idx, W1, b1, W2, b2):
    n_tok, d_model = inputs.shape
    d_ff = W1.shape[1]
    cap, n_exp = top_idx.shape

    idxT = top_idx.astype(jnp.int32).T
    counts = _sc_histogram(idxT)

    blk_m = 512
    grid = (n_tok // blk_m,)
    ratio_scale = 1.0 / (n_exp * cap * d_ff)

    out, ratio = pl.pallas_call(
        functools.partial(_ffn_body, ratio_scale),
        grid=grid,
        in_specs=[
            pl.BlockSpec((blk_m, d_model), lambda i: (i, 0)),
            pl.BlockSpec((d_model, d_ff), lambda i: (0, 0)),
            pl.BlockSpec((1, d_ff), lambda i: (0, 0)),
            pl.BlockSpec((d_ff, d_model), lambda i: (0, 0)),
            pl.BlockSpec((1, d_model), lambda i: (0, 0)),
            pl.BlockSpec((2, n_exp, blk_m), lambda i: (0, 0, i)),
            pl.BlockSpec((blk_m, n_exp), lambda i: (i, 0)),
        ],
        out_specs=[
            pl.BlockSpec((blk_m, d_model), lambda i: (i, 0)),
            pl.BlockSpec((1, 1), lambda i: (0, 0), memory_space=pltpu.SMEM),
        ],
        out_shape=[
            jax.ShapeDtypeStruct((n_tok, d_model), jnp.float32),
            jax.ShapeDtypeStruct((1, 1), jnp.float32),
        ],
    )(inputs, W1, b1.reshape(1, d_ff), W2, b2.reshape(1, d_model), counts,
      inputs_weight)
    return out, ratio[0, 0]
